# Initial kernel scaffold; baseline (speedup 1.0000x reference)
import jax
import jax.numpy as jnp
from jax.experimental import pallas as pl


def kernel(x, edge_index, W1l, b1l, W1r, W2l, b2l, W2r):
    raise NotImplementedError("write your pallas kernel here")


# speedup vs baseline: 9.8514x; 9.8514x over previous
import functools

import jax
import jax.numpy as jnp
from jax import lax
from jax.experimental import pallas as pl
from jax.experimental.pallas import tpu as pltpu
from jax.experimental.pallas import tpu_sc as plsc

NC = 2
NS = 16
NW = NC * NS
L = 16
CHUNK = 128


def _sc_aggregate(feat, src, dst, n_pad, with_cnt):
    d = feat.shape[1]
    e_pad = src.shape[0]
    cpt = e_pad // (NW * CHUNK)
    rpt = n_pad // NS

    def body(feat_hbm, src_hbm, dst_hbm, out_hbm, *rest):
        if with_cnt:
            (cnt_hbm, idx_s0, idx_s1, idx_d0, idx_d1, rows0, rows1,
             sem0, sem1, ones_c, zb_v, acc_sh, cnt_sh) = rest
        else:
            idx_s0, idx_s1, idx_d0, idx_d1, rows0, rows1, sem0, sem1, acc_sh = rest
            cnt_hbm = ones_c = zb_v = cnt_sh = None
        rows_v = rows0
        cid = lax.axis_index("c")
        sid = lax.axis_index("s")
        wid = sid * NC + cid

        zrow = jnp.zeros((L,), jnp.float32)

        def zero_rows(i, _):
            for j in range(d // L):
                rows_v[i, pl.ds(j * L, L)] = zrow
            return 0

        lax.fori_loop(0, CHUNK, zero_rows, 0)
        if with_cnt:
            def fill_ones(i, _):
                ones_c[pl.ds(i * L, L)] = jnp.ones((L,), jnp.float32)
                return 0

            lax.fori_loop(0, CHUNK // L, fill_ones, 0)

            def zero_zb(i, _):
                zb_v[pl.ds(i * L, L)] = zrow
                return 0

            lax.fori_loop(0, zb_v.shape[0] // L, zero_zb, 0)

        for k in range(0, rpt, CHUNK):
            nr = min(CHUNK, rpt - k)
            off = sid * rpt + k
            pltpu.sync_copy(rows_v.at[pl.ds(0, nr)], acc_sh.at[pl.ds(off, nr)])
        if with_cnt:
            pltpu.sync_copy(zb_v.at[pl.ds(0, rpt)],
                            cnt_sh.at[pl.ds(sid * rpt, rpt)])
        plsc.subcore_barrier()

        def issue(j, idx_s, idx_d, rows, sem):
            off = (wid * cpt + j) * CHUNK
            pltpu.sync_copy(src_hbm.at[pl.ds(off, CHUNK)], idx_s)
            pltpu.sync_copy(dst_hbm.at[pl.ds(off, CHUNK)], idx_d)
            pltpu.make_async_copy(feat_hbm.at[idx_s], rows, sem).start()

        def consume(idx_s, idx_d, rows, sem):
            pltpu.make_async_copy(feat_hbm.at[idx_s], rows, sem).wait()
            pltpu.sync_copy(rows, acc_sh.at[idx_d], add=True)
            if with_cnt:
                pltpu.sync_copy(ones_c, cnt_sh.at[idx_d], add=True)

        issue(0, idx_s0, idx_d0, rows0, sem0)

        def step(k, _):
            j1 = 2 * k + 1
            issue(j1, idx_s1, idx_d1, rows1, sem1)
            consume(idx_s0, idx_d0, rows0, sem0)

            @pl.when(j1 + 1 < cpt)
            def _():
                issue(j1 + 1, idx_s0, idx_d0, rows0, sem0)

            consume(idx_s1, idx_d1, rows1, sem1)
            return 0

        lax.fori_loop(0, cpt // 2, step, 0)
        plsc.subcore_barrier()

        pltpu.sync_copy(acc_sh.at[pl.ds(sid * rpt, rpt)],
                        out_hbm.at[cid, pl.ds(sid * rpt, rpt)])
        if with_cnt:
            pltpu.sync_copy(cnt_sh.at[pl.ds(sid * rpt, rpt)],
                            zb_v.at[pl.ds(0, rpt)])
            pltpu.sync_copy(zb_v.at[pl.ds(0, rpt)],
                            cnt_hbm.at[pl.ds(cid * n_pad + sid * rpt, rpt)])

    out_type = [jax.ShapeDtypeStruct((NC, n_pad, d), jnp.float32)]
    scratch = [
        pltpu.VMEM((CHUNK,), jnp.int32),
        pltpu.VMEM((CHUNK,), jnp.int32),
        pltpu.VMEM((CHUNK,), jnp.int32),
        pltpu.VMEM((CHUNK,), jnp.int32),
        pltpu.VMEM((CHUNK, d), jnp.float32),
        pltpu.VMEM((CHUNK, d), jnp.float32),
        pltpu.SemaphoreType.DMA,
        pltpu.SemaphoreType.DMA,
    ]
    if with_cnt:
        out_type.append(jax.ShapeDtypeStruct((NC * n_pad,), jnp.float32))
        scratch.append(pltpu.VMEM((CHUNK,), jnp.float32))
        scratch.append(pltpu.VMEM(((rpt + L - 1) // L * L,), jnp.float32))
    scratch.append(pltpu.VMEM_SHARED((n_pad, d), jnp.float32))
    if with_cnt:
        scratch.append(pltpu.VMEM_SHARED((n_pad,), jnp.float32))

    mesh = plsc.VectorSubcoreMesh(core_axis_name="c", subcore_axis_name="s")
    k = pl.kernel(body, out_type=tuple(out_type), mesh=mesh,
                  scratch_types=tuple(scratch))
    return k(feat, src, dst)


def _tc_sage_body(p_ref, c_ref, x_ref, wl_ref, b_ref, wr_ref, o_ref, *, relu):
    s = p_ref[0] + p_ref[1]
    c = (c_ref[:, 0] + c_ref[:, 1])[:, None]
    mean = s / jnp.maximum(c, 1.0)
    r = (jnp.dot(mean, wl_ref[...], preferred_element_type=jnp.float32,
                 precision=lax.Precision.HIGHEST)
         + b_ref[...]
         + jnp.dot(x_ref[...], wr_ref[...], preferred_element_type=jnp.float32,
                   precision=lax.Precision.HIGHEST))
    o_ref[...] = jnp.maximum(r, 0.0) if relu else r


def _tc_sage(p, cnt, feat, wl_t, bl, wr_t, relu):
    n_pad, d = feat.shape
    blk = n_pad // 16
    grid = (n_pad // blk,)
    return pl.pallas_call(
        functools.partial(_tc_sage_body, relu=relu),
        grid=grid,
        in_specs=[
            pl.BlockSpec((NC, blk, d), lambda i: (0, i, 0)),
            pl.BlockSpec((blk, NC), lambda i: (i, 0)),
            pl.BlockSpec((blk, d), lambda i: (i, 0)),
            pl.BlockSpec((d, d), lambda i: (0, 0)),
            pl.BlockSpec((1, d), lambda i: (0, 0)),
            pl.BlockSpec((d, d), lambda i: (0, 0)),
        ],
        out_specs=pl.BlockSpec((blk, d), lambda i: (i, 0)),
        out_shape=jax.ShapeDtypeStruct((n_pad, d), jnp.float32),
    )(p, cnt, feat, wl_t, bl, wr_t)


def kernel(x, edge_index, W1l, b1l, W1r, W2l, b2l, W2r):
    n, d = x.shape
    e = edge_index.shape[1]

    n_pad = ((n + NS * 8 - 1) // (NS * 8)) * (NS * 8)
    e_pad = ((e + 2 * NW * CHUNK - 1) // (2 * NW * CHUNK)) * (2 * NW * CHUNK)

    src = edge_index[0].astype(jnp.int32)
    dst = edge_index[1].astype(jnp.int32)
    npad_e = e_pad - e
    if npad_e:
        fill = jnp.arange(npad_e, dtype=jnp.int32)
        src = jnp.concatenate([src, fill % n])
        dst = jnp.concatenate([dst, n + fill % (n_pad - n)])
    x_pad = jnp.pad(x, ((0, n_pad - n), (0, 0)))

    w1l_t, w1r_t = W1l.T, W1r.T
    w2l_t, w2r_t = W2l.T, W2r.T
    b1 = b1l.reshape(1, d)
    b2 = b2l.reshape(1, d)

    p1, cnt = _sc_aggregate(x_pad, src, dst, n_pad, with_cnt=True)
    cnt = cnt.reshape(NC, n_pad).T
    h = _tc_sage(p1, cnt, x_pad, w1l_t, b1, w1r_t, relu=True)
    (p2,) = _sc_aggregate(h, src, dst, n_pad, with_cnt=False)
    out = _tc_sage(p2, cnt, h, w2l_t, b2, w2r_t, relu=False)
    return out[:n]
